# baseline (speedup 1.0000x reference)
import functools

import jax
import jax.numpy as jnp
from jax import lax
from jax.experimental import pallas as pl
from jax.experimental.pallas import tpu as pltpu


def _fc_slab(hseq_sc, pidx, w_fc_ref, b_fc_ref, logits_ref, lo, hi):
    _, Tc, _B, H = hseq_sc.shape
    V = w_fc_ref.shape[1]
    nb = hi - lo
    hs = jnp.swapaxes(hseq_sc[pidx, :, lo:hi, :], 0, 1).reshape(nb * Tc, H)
    logits = jnp.dot(hs, w_fc_ref[...],
                     preferred_element_type=jnp.float32) + b_fc_ref[...]
    logits_ref[lo:hi, :, :] = logits.reshape(nb, Tc, V)


def _lstm_kernel(x_ref, w_ih_ref, w_hh_ref, b_ref, w_fc_ref, b_fc_ref,
                 logits_ref, h_out_ref, c_out_ref,
                 h_sc, c_sc, gates_sc, hseq_sc):
    tc = pl.program_id(0)
    n_grid = pl.num_programs(0)
    Tc, B, _E = x_ref.shape
    H = w_hh_ref.shape[0]
    cur = lax.rem(tc, 2)
    prev = 1 - cur
    n_slabs = 4
    slab = B // n_slabs

    @pl.when(tc == 0)
    def _():
        h_sc[...] = jnp.zeros_like(h_sc)
        c_sc[...] = jnp.zeros_like(c_sc)

    @pl.when(tc == n_grid - 1)
    def _():
        h_out_ref[...] = h_sc[...]
        c_out_ref[...] = c_sc[...]

    @pl.when(tc < n_grid - 1)
    def _():
        h, c = h_sc[...], c_sc[...]
        for t in range(Tc):
            buf = t % 2
            gates_x = jnp.dot(x_ref[t], w_ih_ref[...],
                              preferred_element_type=jnp.float32) + b_ref[...]
            gates_sc[buf] = gates_x

            gates = gates_sc[buf] + jnp.dot(
                h.astype(jnp.bfloat16), w_hh_ref[...],
                preferred_element_type=jnp.float32)
            ifo = jax.nn.sigmoid(gates[:, :3 * H])
            g_g = jnp.tanh(gates[:, 3 * H:])
            i_g = ifo[:, 0 * H:1 * H]
            f_g = ifo[:, 1 * H:2 * H]
            o_g = ifo[:, 2 * H:3 * H]
            c = f_g * c + i_g * g_g
            h = o_g * jnp.tanh(c)
            hseq_sc[cur, t] = h.astype(jnp.bfloat16)

            if t % 2 == 1:
                k = t // 2
                _fc_slab(hseq_sc, prev, w_fc_ref, b_fc_ref, logits_ref,
                         k * slab, (k + 1) * slab)

        h_sc[...] = h
        c_sc[...] = c

    @pl.when(tc == n_grid - 1)
    def _():
        for k in range(n_slabs):
            _fc_slab(hseq_sc, prev, w_fc_ref, b_fc_ref, logits_ref,
                     k * slab, (k + 1) * slab)


def _forward(tokens, emb, w_ih, w_hh, b, w_fc, b_fc, *, t_chunk, vmem_mb):
    B, T = tokens.shape
    V, E = emb.shape
    H = w_hh.shape[0]
    n_chunks = T // t_chunk

    x_tm = emb[tokens.T].astype(jnp.bfloat16)

    wconst = lambda t: (0, 0)

    logits, h_n, c_n = pl.pallas_call(
        _lstm_kernel,
        out_shape=(
            jax.ShapeDtypeStruct((B, T, V), jnp.float32),
            jax.ShapeDtypeStruct((B, H), jnp.float32),
            jax.ShapeDtypeStruct((B, H), jnp.float32),
        ),
        grid_spec=pltpu.PrefetchScalarGridSpec(
            num_scalar_prefetch=0,
            grid=(n_chunks + 1,),
            in_specs=[
                pl.BlockSpec((t_chunk, B, E),
                             lambda t: (jnp.minimum(t, T // t_chunk - 1),
                                        0, 0)),
                pl.BlockSpec((E, 4 * H), wconst, pipeline_mode=pl.Buffered(1)),
                pl.BlockSpec((H, 4 * H), wconst, pipeline_mode=pl.Buffered(1)),
                pl.BlockSpec((1, 4 * H), wconst, pipeline_mode=pl.Buffered(1)),
                pl.BlockSpec((H, V), wconst, pipeline_mode=pl.Buffered(1)),
                pl.BlockSpec((1, V), wconst, pipeline_mode=pl.Buffered(1)),
            ],
            out_specs=[
                pl.BlockSpec((B, t_chunk, V),
                             lambda t: (0, jnp.maximum(t - 1, 0), 0)),
                pl.BlockSpec((B, H), lambda t: (0, 0)),
                pl.BlockSpec((B, H), lambda t: (0, 0)),
            ],
            scratch_shapes=[
                pltpu.VMEM((B, H), jnp.float32),
                pltpu.VMEM((B, H), jnp.float32),
                pltpu.VMEM((2, B, 4 * H), jnp.float32),
                pltpu.VMEM((2, t_chunk, B, H), jnp.bfloat16),
            ],
        ),
        compiler_params=pltpu.CompilerParams(
            dimension_semantics=("arbitrary",),
            vmem_limit_bytes=vmem_mb << 20),
    )(x_tm, w_ih, w_hh, b, w_fc, b_fc)

    return logits, (h_n[None, :, :], c_n[None, :, :])


def kernel(tokens, embedding, w_ih, w_hh, b, w_fc, b_fc):
    return _forward(tokens, embedding, w_ih, w_hh, b, w_fc, b_fc,
                    t_chunk=8, vmem_mb=58)


# speedup vs baseline: 1.0406x; 1.0406x over previous
import functools

import jax
import jax.numpy as jnp
from jax import lax
from jax.experimental import pallas as pl
from jax.experimental.pallas import tpu as pltpu


def _lstm_kernel(x_ref, w_ih_ref, w_hh_ref, b_ref, w_fc_ref, b_fc_ref,
                 logits_ref, h_out_ref, c_out_ref,
                 h_sc, c_sc, gates_sc, hseq_sc, *, unroll):
    tc = pl.program_id(0)
    Tc, B, _E = x_ref.shape
    H = w_hh_ref.shape[0]
    V = w_fc_ref.shape[1]

    @pl.when(tc == 0)
    def _():
        h_sc[...] = jnp.zeros_like(h_sc)
        c_sc[...] = jnp.zeros_like(c_sc)

    quart = Tc // 8
    carry = (h_sc[...], c_sc[...])
    for sub in range(8):
        buf = sub % 2
        x_flat = x_ref[sub * quart:(sub + 1) * quart].reshape(
            quart * B, x_ref.shape[2])
        gates_x = jnp.dot(x_flat, w_ih_ref[...],
                          preferred_element_type=jnp.float32) + b_ref[...]
        gates_sc[buf] = gates_x.reshape(quart, B, 4 * H)

        def step(t, c_in, *, base=sub * quart, buf=buf):
            h, c = c_in
            gates = gates_sc[buf, t] + jnp.dot(h.astype(jnp.bfloat16),
                                               w_hh_ref[...],
                                               preferred_element_type=jnp.float32)
            ifo = jax.nn.sigmoid(gates[:, :3 * H])
            g_g = jnp.tanh(gates[:, 3 * H:])
            i_g = ifo[:, 0 * H:1 * H]
            f_g = ifo[:, 1 * H:2 * H]
            o_g = ifo[:, 2 * H:3 * H]
            c_new = f_g * c + i_g * g_g
            h_new = o_g * jnp.tanh(c_new)
            hseq_sc[base + t] = h_new.astype(jnp.bfloat16)
            return (h_new, c_new)

        carry = lax.fori_loop(0, quart, step, carry, unroll=unroll)

    h_fin, c_fin = carry
    h_sc[...] = h_fin
    c_sc[...] = c_fin

    hs_bt = jnp.swapaxes(hseq_sc[...], 0, 1).reshape(B * Tc, H)
    logits = jnp.dot(hs_bt, w_fc_ref[...],
                     preferred_element_type=jnp.float32) + b_fc_ref[...]
    logits_ref[...] = logits.reshape(B, Tc, V)

    @pl.when(tc == pl.num_programs(0) - 1)
    def _():
        h_out_ref[...] = h_fin
        c_out_ref[...] = c_fin


def _forward(tokens, emb, w_ih, w_hh, b, w_fc, b_fc, *, t_chunk, unroll,
             vmem_mb):
    B, T = tokens.shape
    V, E = emb.shape
    H = w_hh.shape[0]
    n_chunks = T // t_chunk

    x_tm = emb[tokens.T].astype(jnp.bfloat16)

    wconst = lambda t: (0, 0)

    logits, h_n, c_n = pl.pallas_call(
        functools.partial(_lstm_kernel, unroll=unroll),
        out_shape=(
            jax.ShapeDtypeStruct((B, T, V), jnp.float32),
            jax.ShapeDtypeStruct((B, H), jnp.float32),
            jax.ShapeDtypeStruct((B, H), jnp.float32),
        ),
        grid_spec=pltpu.PrefetchScalarGridSpec(
            num_scalar_prefetch=0,
            grid=(n_chunks,),
            in_specs=[
                pl.BlockSpec((t_chunk, B, E), lambda t: (t, 0, 0)),
                pl.BlockSpec((E, 4 * H), wconst, pipeline_mode=pl.Buffered(1)),
                pl.BlockSpec((H, 4 * H), wconst, pipeline_mode=pl.Buffered(1)),
                pl.BlockSpec((1, 4 * H), wconst, pipeline_mode=pl.Buffered(1)),
                pl.BlockSpec((H, V), wconst, pipeline_mode=pl.Buffered(1)),
                pl.BlockSpec((1, V), wconst, pipeline_mode=pl.Buffered(1)),
            ],
            out_specs=[
                pl.BlockSpec((B, t_chunk, V), lambda t: (0, t, 0)),
                pl.BlockSpec((B, H), lambda t: (0, 0)),
                pl.BlockSpec((B, H), lambda t: (0, 0)),
            ],
            scratch_shapes=[
                pltpu.VMEM((B, H), jnp.float32),
                pltpu.VMEM((B, H), jnp.float32),
                pltpu.VMEM((2, t_chunk // 8, B, 4 * H),
                           jnp.float32),
                pltpu.VMEM((t_chunk, B, H), jnp.bfloat16),
            ],
        ),
        compiler_params=pltpu.CompilerParams(
            dimension_semantics=("arbitrary",),
            vmem_limit_bytes=vmem_mb << 20),
    )(x_tm, w_ih, w_hh, b, w_fc, b_fc)

    return logits, (h_n[None, :, :], c_n[None, :, :])


def kernel(tokens, embedding, w_ih, w_hh, b, w_fc, b_fc):
    return _forward(tokens, embedding, w_ih, w_hh, b, w_fc, b_fc,
                    t_chunk=8, unroll=4, vmem_mb=58)


# speedup vs baseline: 1.0407x; 1.0002x over previous
import functools

import jax
import jax.numpy as jnp
from jax import lax
from jax.experimental import pallas as pl
from jax.experimental.pallas import tpu as pltpu


def _lstm_kernel(x_ref, w_ih_ref, w_hh_ref, b_ref, w_fc_ref, b_fc_ref,
                 logits_ref, h_out_ref, c_out_ref,
                 h_sc, c_sc, gates_sc, hseq_sc, *, unroll):
    tc = pl.program_id(0)
    Tc, B, _E = x_ref.shape
    H = w_hh_ref.shape[0]
    V = w_fc_ref.shape[1]

    @pl.when(tc == 0)
    def _():
        h_sc[...] = jnp.zeros_like(h_sc)
        c_sc[...] = jnp.zeros_like(c_sc)

    quart = Tc // 8
    carry = (h_sc[...], c_sc[...])
    for sub in range(8):
        buf = sub % 3
        x_flat = x_ref[sub * quart:(sub + 1) * quart].reshape(
            quart * B, x_ref.shape[2])
        gates_x = jnp.dot(x_flat, w_ih_ref[...],
                          preferred_element_type=jnp.float32) + b_ref[...]
        gates_sc[buf] = gates_x.reshape(quart, B, 4 * H)

        def step(t, c_in, *, base=sub * quart, buf=buf):
            h, c = c_in
            gates = gates_sc[buf, t] + jnp.dot(h.astype(jnp.bfloat16),
                                               w_hh_ref[...],
                                               preferred_element_type=jnp.float32)
            ifo = jax.nn.sigmoid(gates[:, :3 * H])
            g_g = jnp.tanh(gates[:, 3 * H:])
            i_g = ifo[:, 0 * H:1 * H]
            f_g = ifo[:, 1 * H:2 * H]
            o_g = ifo[:, 2 * H:3 * H]
            c_new = f_g * c + i_g * g_g
            h_new = o_g * jnp.tanh(c_new)
            hseq_sc[base + t] = h_new.astype(jnp.bfloat16)
            return (h_new, c_new)

        carry = lax.fori_loop(0, quart, step, carry, unroll=unroll)

    h_fin, c_fin = carry
    h_sc[...] = h_fin
    c_sc[...] = c_fin

    hs_bt = jnp.swapaxes(hseq_sc[...], 0, 1).reshape(B * Tc, H)
    logits = jnp.dot(hs_bt, w_fc_ref[...],
                     preferred_element_type=jnp.float32) + b_fc_ref[...]
    logits_ref[...] = logits.reshape(B, Tc, V)

    @pl.when(tc == pl.num_programs(0) - 1)
    def _():
        h_out_ref[...] = h_fin
        c_out_ref[...] = c_fin


def _forward(tokens, emb, w_ih, w_hh, b, w_fc, b_fc, *, t_chunk, unroll,
             vmem_mb):
    B, T = tokens.shape
    V, E = emb.shape
    H = w_hh.shape[0]
    n_chunks = T // t_chunk

    x_tm = emb[tokens.T].astype(jnp.bfloat16)

    wconst = lambda t: (0, 0)

    logits, h_n, c_n = pl.pallas_call(
        functools.partial(_lstm_kernel, unroll=unroll),
        out_shape=(
            jax.ShapeDtypeStruct((B, T, V), jnp.float32),
            jax.ShapeDtypeStruct((B, H), jnp.float32),
            jax.ShapeDtypeStruct((B, H), jnp.float32),
        ),
        grid_spec=pltpu.PrefetchScalarGridSpec(
            num_scalar_prefetch=0,
            grid=(n_chunks,),
            in_specs=[
                pl.BlockSpec((t_chunk, B, E), lambda t: (t, 0, 0)),
                pl.BlockSpec((E, 4 * H), wconst, pipeline_mode=pl.Buffered(1)),
                pl.BlockSpec((H, 4 * H), wconst, pipeline_mode=pl.Buffered(1)),
                pl.BlockSpec((1, 4 * H), wconst, pipeline_mode=pl.Buffered(1)),
                pl.BlockSpec((H, V), wconst, pipeline_mode=pl.Buffered(1)),
                pl.BlockSpec((1, V), wconst, pipeline_mode=pl.Buffered(1)),
            ],
            out_specs=[
                pl.BlockSpec((B, t_chunk, V), lambda t: (0, t, 0)),
                pl.BlockSpec((B, H), lambda t: (0, 0)),
                pl.BlockSpec((B, H), lambda t: (0, 0)),
            ],
            scratch_shapes=[
                pltpu.VMEM((B, H), jnp.float32),
                pltpu.VMEM((B, H), jnp.float32),
                pltpu.VMEM((3, t_chunk // 8, B, 4 * H),
                           jnp.float32),
                pltpu.VMEM((t_chunk, B, H), jnp.bfloat16),
            ],
        ),
        compiler_params=pltpu.CompilerParams(
            dimension_semantics=("arbitrary",),
            vmem_limit_bytes=vmem_mb << 20),
    )(x_tm, w_ih, w_hh, b, w_fc, b_fc)

    return logits, (h_n[None, :, :], c_n[None, :, :])


def kernel(tokens, embedding, w_ih, w_hh, b, w_fc, b_fc):
    return _forward(tokens, embedding, w_ih, w_hh, b, w_fc, b_fc,
                    t_chunk=8, unroll=4, vmem_mb=58)
